# baseline (speedup 1.0000x reference)
import jax
import jax.numpy as jnp
from jax import lax
from jax.experimental import pallas as pl
from jax.experimental.pallas import tpu as pltpu
from jax.experimental.pallas import tpu_sc as plsc

_B = 4
_N = 4096
_NW = 32
_RPW = 512
_WPB = 8
_L = 16
_NCH = _N // _L


def _sc_body(x_hbm, y_hbm, rowsum_hbm, colmin_hbm,
             xb0, xb1, xb2, nxv,
             yb0, yb1, yb2, nyv,
             colv, rsv):
    c = lax.axis_index("c")
    s = lax.axis_index("s")
    w = s * 2 + c
    b = w // _WPB
    r0 = (w % _WPB) * _RPW

    xbase = b * 3 * _N + r0
    ybase = b * 3 * _N
    pltpu.sync_copy(x_hbm.at[pl.ds(xbase, _RPW)], xb0.at[pl.ds(0, _RPW)])
    pltpu.sync_copy(x_hbm.at[pl.ds(xbase + _N, _RPW)], xb1.at[pl.ds(0, _RPW)])
    pltpu.sync_copy(x_hbm.at[pl.ds(xbase + 2 * _N, _RPW)], xb2.at[pl.ds(0, _RPW)])
    pltpu.sync_copy(y_hbm.at[pl.ds(ybase, _N)], yb0)
    pltpu.sync_copy(y_hbm.at[pl.ds(ybase + _N, _N)], yb1)
    pltpu.sync_copy(y_hbm.at[pl.ds(ybase + 2 * _N, _N)], yb2)

    f32 = jnp.float32

    def bf16_round(v):
        bits = lax.bitcast_convert_type(v, jnp.int32)
        bits = bits + jnp.int32(0x7FFF) + ((bits >> 16) & jnp.int32(1))
        bits = bits & jnp.int32(-65536)
        return lax.bitcast_convert_type(bits, f32)

    def prep_col(i, carry):
        sl = pl.ds(i * _L, _L)
        v0 = yb0[sl]
        v1 = yb1[sl]
        v2 = yb2[sl]
        nyv[sl] = v0 * v0 + v1 * v1 + v2 * v2
        yb0[sl] = bf16_round(v0)
        yb1[sl] = bf16_round(v1)
        yb2[sl] = bf16_round(v2)
        colv[sl] = jnp.full((_L,), jnp.inf, dtype=f32)
        return carry

    lax.fori_loop(0, _NCH, prep_col, 0, unroll=4)

    def prep_row(i, carry):
        sl = pl.ds(i * _L, _L)
        v0 = xb0[sl]
        v1 = xb1[sl]
        v2 = xb2[sl]
        nxv[sl] = v0 * v0 + v1 * v1 + v2 * v2
        xb0[sl] = bf16_round(v0)
        xb1[sl] = bf16_round(v1)
        xb2[sl] = bf16_round(v2)
        return carry

    lax.fori_loop(0, _RPW // _L, prep_row, 0, unroll=4)

    xb0[pl.ds(_RPW, _L)] = jnp.zeros((_L,), dtype=f32)
    xb1[pl.ds(_RPW, _L)] = jnp.zeros((_L,), dtype=f32)
    xb2[pl.ds(_RPW, _L)] = jnp.zeros((_L,), dtype=f32)
    nxv[pl.ds(_RPW, _L)] = jnp.zeros((_L,), dtype=f32)

    def row_step(i, row_sum):
        bx0 = xb0[pl.ds(i, _L)][0]
        bx1 = xb1[pl.ds(i, _L)][0]
        bx2 = xb2[pl.ds(i, _L)][0]
        nx = nxv[pl.ds(i, _L)][0]

        def col_step(j, racc):
            sl = pl.ds(j * _L, _L)
            t = yb0[sl] * bx0
            t = t + yb1[sl] * bx1
            t = t + yb2[sl] * bx2
            e = nyv[sl] - 2.0 * t
            racc = jnp.minimum(racc, e)
            colv[sl] = jnp.minimum(colv[sl], e + nx)
            return racc

        racc = lax.fori_loop(
            0, _NCH, col_step,
            jnp.full((_L,), jnp.inf, dtype=f32), unroll=4,
        )
        dnums = lax.GatherDimensionNumbers(
            offset_dims=(), collapsed_slice_dims=(0,), start_index_map=(0,)
        )
        for sh in (8, 4, 2, 1):
            idx = (lax.iota(jnp.int32, _L) ^ sh).reshape(_L, 1)
            perm = lax.gather(
                racc, idx, dimension_numbers=dnums, slice_sizes=(1,),
                mode=lax.GatherScatterMode.PROMISE_IN_BOUNDS,
            )
            racc = jnp.minimum(racc, perm)
        return row_sum + (nx + racc[0])

    row_sum = lax.fori_loop(0, _RPW, row_step, jnp.float32(0.0))

    lane0 = lax.iota(jnp.int32, _L) == 0
    rsv[...] = jnp.where(lane0, row_sum, jnp.float32(0.0))
    pltpu.sync_copy(rsv, rowsum_hbm.at[pl.ds(w * _L, _L)])
    pltpu.sync_copy(colv, colmin_hbm.at[pl.ds(w * _N, _N)])


def _finish_body(rowsum_ref, colmin_ref, out_ref):
    total = jnp.sum(rowsum_ref[...])
    for b in range(_B):
        part = colmin_ref[b * _WPB * _N:(b * _WPB + 1) * _N]
        for k in range(1, _WPB):
            lo = (b * _WPB + k) * _N
            part = jnp.minimum(part, colmin_ref[lo:lo + _N])
        total = total + jnp.sum(part)
    out_ref[0, 0] = total * jnp.float32(1.0 / (2.0 * _B * _N))


def kernel(in_pc, target_pc):
    mesh = plsc.VectorSubcoreMesh(core_axis_name="c", subcore_axis_name="s")
    rowsum, colmin = pl.kernel(
        _sc_body,
        mesh=mesh,
        out_type=[
            jax.ShapeDtypeStruct((_NW * _L,), jnp.float32),
            jax.ShapeDtypeStruct((_NW * _N,), jnp.float32),
        ],
        scratch_types=[
            pltpu.VMEM((_RPW + _L,), jnp.float32),
            pltpu.VMEM((_RPW + _L,), jnp.float32),
            pltpu.VMEM((_RPW + _L,), jnp.float32),
            pltpu.VMEM((_RPW + _L,), jnp.float32),
            pltpu.VMEM((_N,), jnp.float32),
            pltpu.VMEM((_N,), jnp.float32),
            pltpu.VMEM((_N,), jnp.float32),
            pltpu.VMEM((_N,), jnp.float32),
            pltpu.VMEM((_N,), jnp.float32),
            pltpu.VMEM((_L,), jnp.float32),
        ],
    )(in_pc.reshape(-1), target_pc.reshape(-1))

    total = pl.pallas_call(
        _finish_body,
        out_specs=pl.BlockSpec(memory_space=pltpu.SMEM),
        out_shape=jax.ShapeDtypeStruct((1, 1), jnp.float32),
    )(rowsum, colmin)
    return total[0, 0]


# speedup vs baseline: 33.8791x; 33.8791x over previous
import jax
import jax.numpy as jnp
from jax.experimental import pallas as pl
from jax.experimental.pallas import tpu as pltpu

_B = 4
_N = 4096
_ROW_TILE = 1024


def _chamfer_body(x_ref, y_ref, out_ref):
    total = jnp.float32(0.0)
    for b in range(_B):
        x = x_ref[b]
        y = y_ref[b]

        nx = x[0:1, :] ** 2 + x[1:2, :] ** 2 + x[2:3, :] ** 2
        nxhi = nx.astype(jnp.bfloat16)
        nxlo = (nx - nxhi.astype(jnp.float32)).astype(jnp.bfloat16)
        a_aug = jnp.concatenate(
            [
                x.astype(jnp.bfloat16),
                nxhi,
                nxlo,
                jnp.ones((2, _N), dtype=jnp.bfloat16),
            ],
            axis=0,
        )

        ny = y[0:1, :] ** 2 + y[1:2, :] ** 2 + y[2:3, :] ** 2
        nyhi = ny.astype(jnp.bfloat16)
        nylo = (ny - nyhi.astype(jnp.float32)).astype(jnp.bfloat16)
        b_aug = jnp.concatenate(
            [
                jnp.bfloat16(-2.0) * y.astype(jnp.bfloat16),
                jnp.ones((2, _N), dtype=jnp.bfloat16),
                nyhi,
                nylo,
            ],
            axis=0,
        )

        row_sum = jnp.float32(0.0)
        col_min = jnp.full((1, _N), jnp.inf, dtype=jnp.float32)
        for t in range(_N // _ROW_TILE):
            lo = t * _ROW_TILE
            dist = jax.lax.dot_general(
                a_aug[:, lo:lo + _ROW_TILE], b_aug,
                dimension_numbers=(((0,), (0,)), ((), ())),
                preferred_element_type=jnp.float32,
            )
            row_min = jnp.min(dist, axis=1)
            row_sum = row_sum + jnp.sum(row_min)
            col_min = jnp.minimum(
                col_min, jnp.min(dist, axis=0, keepdims=True)
            )
        total = total + row_sum + jnp.sum(col_min)

    out_ref[0, 0] = total * jnp.float32(1.0 / (2.0 * _B * _N))


def kernel(in_pc, target_pc):
    total = pl.pallas_call(
        _chamfer_body,
        out_specs=pl.BlockSpec(memory_space=pltpu.SMEM),
        out_shape=jax.ShapeDtypeStruct((1, 1), jnp.float32),
    )(in_pc, target_pc)
    return total[0, 0]


# speedup vs baseline: 33.9322x; 1.0016x over previous
import jax
import jax.numpy as jnp
from jax.experimental import pallas as pl
from jax.experimental.pallas import tpu as pltpu

_B = 4
_N = 4096
_ROW_TILE = 2048


def _chamfer_body(x_ref, y_ref, out_ref):
    total = jnp.float32(0.0)
    for b in range(_B):
        x = x_ref[b]
        y = y_ref[b]

        nx = x[0:1, :] ** 2 + x[1:2, :] ** 2 + x[2:3, :] ** 2
        nxhi = nx.astype(jnp.bfloat16)
        nxlo = (nx - nxhi.astype(jnp.float32)).astype(jnp.bfloat16)
        a_aug = jnp.concatenate(
            [
                x.astype(jnp.bfloat16),
                nxhi,
                nxlo,
                jnp.ones((2, _N), dtype=jnp.bfloat16),
            ],
            axis=0,
        )

        ny = y[0:1, :] ** 2 + y[1:2, :] ** 2 + y[2:3, :] ** 2
        nyhi = ny.astype(jnp.bfloat16)
        nylo = (ny - nyhi.astype(jnp.float32)).astype(jnp.bfloat16)
        b_aug = jnp.concatenate(
            [
                jnp.bfloat16(-2.0) * y.astype(jnp.bfloat16),
                jnp.ones((2, _N), dtype=jnp.bfloat16),
                nyhi,
                nylo,
            ],
            axis=0,
        )

        row_sum = jnp.float32(0.0)
        col_min = jnp.full((1, _N), jnp.inf, dtype=jnp.float32)
        for t in range(_N // _ROW_TILE):
            lo = t * _ROW_TILE
            dist = jax.lax.dot_general(
                a_aug[:, lo:lo + _ROW_TILE], b_aug,
                dimension_numbers=(((0,), (0,)), ((), ())),
                preferred_element_type=jnp.float32,
            )
            row_min = jnp.min(dist, axis=1)
            row_sum = row_sum + jnp.sum(row_min)
            col_min = jnp.minimum(
                col_min, jnp.min(dist, axis=0, keepdims=True)
            )
        total = total + row_sum + jnp.sum(col_min)

    out_ref[0, 0] = total * jnp.float32(1.0 / (2.0 * _B * _N))


def kernel(in_pc, target_pc):
    total = pl.pallas_call(
        _chamfer_body,
        out_specs=pl.BlockSpec(memory_space=pltpu.SMEM),
        out_shape=jax.ShapeDtypeStruct((1, 1), jnp.float32),
    )(in_pc, target_pc)
    return total[0, 0]
